# Initial kernel scaffold; baseline (speedup 1.0000x reference)
import jax
import jax.numpy as jnp
from jax.experimental import pallas as pl


def kernel(indices, features, ks_w, ks_b, qs_w, qs_b):
    raise NotImplementedError("write your pallas kernel here")


# speedup vs baseline: 21.5223x; 21.5223x over previous
import functools
import math

import jax
import jax.numpy as jnp
from jax import lax
from jax.experimental import pallas as pl
from jax.experimental.pallas import tpu as pltpu
from jax.experimental.pallas import tpu_sc as plsc

B, N, K, D_LAT, D_KQ = 2, 10000, 64, 128, 32
P = N * K
NW = 32
CHUNK = 1024
SUB = 128
NSUB = CHUNK // SUB
NCHUNK = 20
PAIRS_PER_W = CHUNK * NCHUNK
P_PAD = NW * PAIRS_PER_W
ROWS_PER_W = PAIRS_PER_W // SUB
SCALE = 1.0 / math.sqrt(D_KQ)


def _proj_body(f_ref, kw_ref, kb_ref, qw_ref, qb_ref, ko_ref, qo_ref):
    f = f_ref[0]
    ko_ref[0] = jnp.dot(f, kw_ref[...], preferred_element_type=jnp.float32) + kb_ref[...]
    qo_ref[0] = jnp.dot(f, qw_ref[...], preferred_element_type=jnp.float32) + qb_ref[...]


def _project(features, ks_w, ks_b, qs_w, qs_b):
    R = 2000
    grid = (B, N // R)
    ks_tab, qs_tab = pl.pallas_call(
        _proj_body,
        grid=grid,
        in_specs=[
            pl.BlockSpec((1, R, D_LAT), lambda b, i: (b, i, 0)),
            pl.BlockSpec((D_LAT, D_KQ), lambda b, i: (0, 0)),
            pl.BlockSpec((1, D_KQ), lambda b, i: (0, 0)),
            pl.BlockSpec((D_LAT, D_KQ), lambda b, i: (0, 0)),
            pl.BlockSpec((1, D_KQ), lambda b, i: (0, 0)),
        ],
        out_specs=[
            pl.BlockSpec((1, R, D_KQ), lambda b, i: (b, i, 0)),
            pl.BlockSpec((1, R, D_KQ), lambda b, i: (b, i, 0)),
        ],
        out_shape=[
            jax.ShapeDtypeStruct((B, N, D_KQ), jnp.float32),
            jax.ShapeDtypeStruct((B, N, D_KQ), jnp.float32),
        ],
    )(features, ks_w.T, ks_b.reshape(1, D_KQ), qs_w.T, qs_b.reshape(1, D_KQ))
    return ks_tab, qs_tab


def _sc_affinity(xidx, yidx, ks0, ks1, qs0, qs1):
    info = plsc.get_sparse_core_info()
    nc = info.num_cores

    mesh = plsc.VectorSubcoreMesh(core_axis_name="c", subcore_axis_name="s")

    @functools.partial(
        pl.kernel,
        mesh=mesh,
        out_type=jax.ShapeDtypeStruct((B, P_PAD), jnp.float32),
        compiler_params=pltpu.CompilerParams(
            needs_layout_passes=False, use_tc_tiling_on_sc=False),
        scratch_types=[
            pltpu.VMEM((NSUB, SUB), jnp.int32),
            pltpu.VMEM((NSUB, SUB), jnp.int32),
            pltpu.VMEM((CHUNK, D_KQ), jnp.float32),
            pltpu.VMEM((CHUNK, D_KQ), jnp.float32),
            pltpu.VMEM((CHUNK,), jnp.float32),
            pltpu.SemaphoreType.DMA,
            pltpu.SemaphoreType.DMA,
        ],
    )
    def body(xidx_hbm, yidx_hbm, ks0_hbm, ks1_hbm, qs0_hbm, qs1_hbm, out_hbm,
             xidx_v, yidx_v, xrows_v, yrows_v, out_v, sem_x, sem_y):
        wid = lax.axis_index("s") * nc + lax.axis_index("c")

        for b, kst, qst in ((0, ks0_hbm, qs0_hbm), (1, ks1_hbm, qs1_hbm)):

            def chunk_body(j, carry):
                rbase = wid * ROWS_PER_W + j * NSUB
                pltpu.sync_copy(xidx_hbm.at[b].at[pl.ds(rbase, NSUB)], xidx_v)
                pltpu.sync_copy(yidx_hbm.at[b].at[pl.ds(rbase, NSUB)], yidx_v)
                copies = []
                for s in range(NSUB):
                    copies.append(pltpu.async_copy(
                        kst.at[xidx_v.at[s]],
                        xrows_v.at[pl.ds(s * SUB, SUB)], sem_x))
                for s in range(NSUB):
                    copies.append(pltpu.async_copy(
                        qst.at[yidx_v.at[s]],
                        yrows_v.at[pl.ds(s * SUB, SUB)], sem_y))
                for c in copies:
                    c.wait()

                def group_body(g, carry2):
                    pids = lax.iota(jnp.int32, 16) + g * 16
                    acc = jnp.zeros((16,), jnp.float32)
                    for d in range(D_KQ):
                        dv = jnp.full((16,), d, jnp.int32)
                        xv = plsc.load_gather(xrows_v, [pids, dv])
                        yv = plsc.load_gather(yrows_v, [pids, dv])
                        acc = acc + xv * yv
                    off = pl.multiple_of(g * 16, 16)
                    out_v[pl.ds(off, 16)] = acc * SCALE
                    return carry2

                lax.fori_loop(0, CHUNK // 16, group_body, 0)

                pbase = wid * PAIRS_PER_W + j * CHUNK
                pltpu.sync_copy(out_v, out_hbm.at[b].at[pl.ds(pbase, CHUNK)])
                return carry

            lax.fori_loop(0, NCHUNK, chunk_body, 0)

    return body(xidx, yidx, ks0, ks1, qs0, qs1)


def kernel(indices, features, ks_w, ks_b, qs_w, qs_b):
    ks_tab, qs_tab = _project(features, ks_w, ks_b, qs_w, qs_b)
    pad = ((0, 0), (0, P_PAD - P))
    xidx = jnp.pad(indices[1].reshape(B, P), pad).reshape(B, P_PAD // SUB, SUB)
    yidx = jnp.pad(indices[2].reshape(B, P), pad).reshape(B, P_PAD // SUB, SUB)
    out = _sc_affinity(xidx, yidx, ks_tab[0], ks_tab[1], qs_tab[0], qs_tab[1])
    return out[:, :P].reshape(B, N, K)


# speedup vs baseline: 21.5233x; 1.0000x over previous
import functools
import math

import jax
import jax.numpy as jnp
from jax import lax
from jax.experimental import pallas as pl
from jax.experimental.pallas import tpu as pltpu
from jax.experimental.pallas import tpu_sc as plsc

B, N, K, D_LAT, D_KQ = 2, 10000, 64, 128, 32
P = N * K
NW = 32
CHUNK = 1024
SUB = 128
NSUB = CHUNK // SUB
NCHUNK = 20
PAIRS_PER_W = CHUNK * NCHUNK
P_PAD = NW * PAIRS_PER_W
ROWS_PER_W = PAIRS_PER_W // SUB
SCALE = 1.0 / math.sqrt(D_KQ)


def _proj_body(f_ref, kw_ref, kb_ref, qw_ref, qb_ref, ko_ref, qo_ref):
    f = f_ref[0]
    ko_ref[0] = jnp.dot(f, kw_ref[...], preferred_element_type=jnp.float32) + kb_ref[...]
    qo_ref[0] = jnp.dot(f, qw_ref[...], preferred_element_type=jnp.float32) + qb_ref[...]


def _project(features, ks_w, ks_b, qs_w, qs_b):
    R = 2000
    grid = (B, N // R)
    ks_tab, qs_tab = pl.pallas_call(
        _proj_body,
        grid=grid,
        in_specs=[
            pl.BlockSpec((1, R, D_LAT), lambda b, i: (b, i, 0)),
            pl.BlockSpec((D_LAT, D_KQ), lambda b, i: (0, 0)),
            pl.BlockSpec((1, D_KQ), lambda b, i: (0, 0)),
            pl.BlockSpec((D_LAT, D_KQ), lambda b, i: (0, 0)),
            pl.BlockSpec((1, D_KQ), lambda b, i: (0, 0)),
        ],
        out_specs=[
            pl.BlockSpec((1, R, D_KQ), lambda b, i: (b, i, 0)),
            pl.BlockSpec((1, R, D_KQ), lambda b, i: (b, i, 0)),
        ],
        out_shape=[
            jax.ShapeDtypeStruct((B, N, D_KQ), jnp.float32),
            jax.ShapeDtypeStruct((B, N, D_KQ), jnp.float32),
        ],
    )(features, ks_w.T, ks_b.reshape(1, D_KQ), qs_w.T, qs_b.reshape(1, D_KQ))
    return ks_tab, qs_tab


def _sc_affinity(xidx, yidx, ks0, ks1, qs0, qs1):
    info = plsc.get_sparse_core_info()
    nc = info.num_cores

    mesh = plsc.VectorSubcoreMesh(core_axis_name="c", subcore_axis_name="s")

    @functools.partial(
        pl.kernel,
        mesh=mesh,
        out_type=jax.ShapeDtypeStruct((B, P_PAD), jnp.float32),
        compiler_params=pltpu.CompilerParams(
            needs_layout_passes=False, use_tc_tiling_on_sc=False),
        scratch_types=[
            pltpu.VMEM((NSUB, SUB), jnp.int32),
            pltpu.VMEM((NSUB, SUB), jnp.int32),
            pltpu.VMEM((CHUNK, D_KQ), jnp.float32),
            pltpu.VMEM((CHUNK, D_KQ), jnp.float32),
            pltpu.VMEM((CHUNK,), jnp.float32),
            pltpu.SemaphoreType.DMA,
            pltpu.SemaphoreType.DMA,
        ],
    )
    def body(xidx_hbm, yidx_hbm, ks0_hbm, ks1_hbm, qs0_hbm, qs1_hbm, out_hbm,
             xidx_v, yidx_v, xrows_v, yrows_v, out_v, sem_x, sem_y):
        wid = lax.axis_index("s") * nc + lax.axis_index("c")

        for b, kst, qst in ((0, ks0_hbm, qs0_hbm), (1, ks1_hbm, qs1_hbm)):

            def chunk_body(j, carry):
                rbase = wid * ROWS_PER_W + j * NSUB
                pltpu.sync_copy(xidx_hbm.at[b].at[pl.ds(rbase, NSUB)], xidx_v)
                pltpu.sync_copy(yidx_hbm.at[b].at[pl.ds(rbase, NSUB)], yidx_v)
                copies = []
                for s in range(NSUB):
                    copies.append(pltpu.async_copy(
                        kst.at[xidx_v.at[s]],
                        xrows_v.at[pl.ds(s * SUB, SUB)], sem_x))
                for s in range(NSUB):
                    copies.append(pltpu.async_copy(
                        qst.at[yidx_v.at[s]],
                        yrows_v.at[pl.ds(s * SUB, SUB)], sem_y))
                for c in copies:
                    c.wait()

                def group_body(g, carry2):
                    pids = lax.iota(jnp.int32, 16) + g * 16
                    acc = jnp.zeros((16,), jnp.float32)
                    for d in range(D_KQ):
                        dv = jnp.full((16,), d, jnp.int32)
                        xvv = plsc.load_gather(xrows_v, [pids, dv])
                        yvv = plsc.load_gather(yrows_v, [pids, dv])
                        acc = acc + xvv * yvv
                    off = pl.multiple_of(g * 16, 16)
                    out_v[pl.ds(off, 16)] = acc * SCALE
                    return carry2

                lax.fori_loop(0, CHUNK // 16, group_body, 0)

                pbase = wid * PAIRS_PER_W + j * CHUNK
                pltpu.sync_copy(out_v, out_hbm.at[b].at[pl.ds(pbase, CHUNK)])
                return carry

            lax.fori_loop(0, NCHUNK, chunk_body, 0)

    return body(xidx, yidx, ks0, ks1, qs0, qs1)


def kernel(indices, features, ks_w, ks_b, qs_w, qs_b):
    ks_tab, qs_tab = _project(features, ks_w, ks_b, qs_w, qs_b)
    pad = ((0, 0), (0, P_PAD - P))
    xidx = jnp.pad(indices[1].reshape(B, P), pad).reshape(B, P_PAD // SUB, SUB)
    yidx = jnp.pad(indices[2].reshape(B, P), pad).reshape(B, P_PAD // SUB, SUB)
    out = _sc_affinity(xidx, yidx, ks_tab[0], ks_tab[1], qs_tab[0], qs_tab[1])
    return out[:, :P].reshape(B, N, K)


# speedup vs baseline: 25.8682x; 1.2019x over previous
import functools
import math

import jax
import jax.numpy as jnp
from jax import lax
from jax.experimental import pallas as pl
from jax.experimental.pallas import tpu as pltpu
from jax.experimental.pallas import tpu_sc as plsc

B, N, K, D_LAT, D_KQ = 2, 10000, 64, 128, 32
P = N * K
NW = 32
CHUNK = 1024
SUB = 128
NSUB = CHUNK // SUB
NCHUNK = 20
PAIRS_PER_W = CHUNK * NCHUNK
P_PAD = NW * PAIRS_PER_W
ROWS_PER_W = PAIRS_PER_W // SUB
SCALE = 1.0 / math.sqrt(D_KQ)


def _proj_body(f_ref, kw_ref, kb_ref, qw_ref, qb_ref, ko_ref, qo_ref):
    f = f_ref[0]
    ko_ref[0] = jnp.dot(f, kw_ref[...], preferred_element_type=jnp.float32) + kb_ref[...]
    qo_ref[0] = jnp.dot(f, qw_ref[...], preferred_element_type=jnp.float32) + qb_ref[...]


def _project(features, ks_w, ks_b, qs_w, qs_b):
    R = 2000
    grid = (B, N // R)
    ks_tab, qs_tab = pl.pallas_call(
        _proj_body,
        grid=grid,
        in_specs=[
            pl.BlockSpec((1, R, D_LAT), lambda b, i: (b, i, 0)),
            pl.BlockSpec((D_LAT, D_KQ), lambda b, i: (0, 0)),
            pl.BlockSpec((1, D_KQ), lambda b, i: (0, 0)),
            pl.BlockSpec((D_LAT, D_KQ), lambda b, i: (0, 0)),
            pl.BlockSpec((1, D_KQ), lambda b, i: (0, 0)),
        ],
        out_specs=[
            pl.BlockSpec((1, R, D_KQ), lambda b, i: (b, i, 0)),
            pl.BlockSpec((1, R, D_KQ), lambda b, i: (b, i, 0)),
        ],
        out_shape=[
            jax.ShapeDtypeStruct((B, N, D_KQ), jnp.float32),
            jax.ShapeDtypeStruct((B, N, D_KQ), jnp.float32),
        ],
    )(features, ks_w.T, ks_b.reshape(1, D_KQ), qs_w.T, qs_b.reshape(1, D_KQ))
    return ks_tab, qs_tab


def _sc_affinity(xidx, yidx, ks0, ks1, qs0, qs1):
    info = plsc.get_sparse_core_info()
    nc = info.num_cores

    mesh = plsc.VectorSubcoreMesh(core_axis_name="c", subcore_axis_name="s")

    @functools.partial(
        pl.kernel,
        mesh=mesh,
        out_type=jax.ShapeDtypeStruct((B, P_PAD), jnp.float32),
        compiler_params=pltpu.CompilerParams(
            needs_layout_passes=False, use_tc_tiling_on_sc=False),
        scratch_types=[
            pltpu.VMEM((NSUB, SUB), jnp.int32),
            pltpu.VMEM((NSUB, SUB), jnp.int32),
            pltpu.VMEM((CHUNK, D_KQ), jnp.float32),
            pltpu.VMEM((CHUNK, D_KQ), jnp.float32),
            pltpu.VMEM((CHUNK,), jnp.float32),
            pltpu.SemaphoreType.DMA,
            pltpu.SemaphoreType.DMA,
        ],
    )
    def body(xidx_hbm, yidx_hbm, ks0_hbm, ks1_hbm, qs0_hbm, qs1_hbm, out_hbm,
             xidx_v, yidx_v, xrows_v, yrows_v, out_v, sem_x, sem_y):
        wid = lax.axis_index("s") * nc + lax.axis_index("c")

        for b, kst, qst in ((0, ks0_hbm, qs0_hbm), (1, ks1_hbm, qs1_hbm)):

            def chunk_body(j, carry):
                rbase = wid * ROWS_PER_W + j * NSUB
                pltpu.sync_copy(xidx_hbm.at[b].at[pl.ds(rbase, NSUB)], xidx_v)
                pltpu.sync_copy(yidx_hbm.at[b].at[pl.ds(rbase, NSUB)], yidx_v)
                copies = []
                for s in range(NSUB):
                    copies.append(pltpu.async_copy(
                        kst.at[xidx_v.at[s]],
                        xrows_v.at[pl.ds(s * SUB, SUB)], sem_x))
                for s in range(NSUB):
                    copies.append(pltpu.async_copy(
                        qst.at[yidx_v.at[s]],
                        yrows_v.at[pl.ds(s * SUB, SUB)], sem_y))
                for c in copies:
                    c.wait()

                def group_body(g, carry2):
                    pids = lax.iota(jnp.int32, 16) + g * 16
                    acc = jnp.zeros((16,), jnp.float32)
                    for d in range(D_KQ):
                        dv = jnp.full((16,), d, jnp.int32)
                        xvv = plsc.load_gather(xrows_v, [pids, dv])
                        yvv = plsc.load_gather(yrows_v, [pids, dv])
                        acc = acc + xvv * yvv
                    off = pl.multiple_of(g * 16, 16)
                    out_v[pl.ds(off, 16)] = acc * SCALE
                    return carry2

                lax.fori_loop(0, CHUNK // 16, group_body, 0)

                pbase = wid * PAIRS_PER_W + j * CHUNK
                pltpu.sync_copy(out_v, out_hbm.at[b].at[pl.ds(pbase, CHUNK)])
                return carry

            lax.fori_loop(0, NCHUNK, chunk_body, 0)

    return body(xidx, yidx, ks0, ks1, qs0, qs1)


def kernel(indices, features, ks_w, ks_b, qs_w, qs_b):
    ks_tab, qs_tab = _project(features, ks_w, ks_b, qs_w, qs_b)
    fill = jax.lax.broadcasted_iota(jnp.int32, (B, P_PAD - P), 1) % N
    xidx = jnp.concatenate(
        [indices[1].reshape(B, P).astype(jnp.int32), fill],
        axis=1).reshape(B, P_PAD // SUB, SUB)
    yidx = jnp.concatenate(
        [indices[2].reshape(B, P).astype(jnp.int32), fill],
        axis=1).reshape(B, P_PAD // SUB, SUB)
    out = _sc_affinity(xidx, yidx, ks_tab[0], ks_tab[1], qs_tab[0], qs_tab[1])
    return out[:, :P].reshape(B, N, K)


# speedup vs baseline: 26.3144x; 1.0172x over previous
import functools
import math

import jax
import jax.numpy as jnp
from jax import lax
from jax.experimental import pallas as pl
from jax.experimental.pallas import tpu as pltpu
from jax.experimental.pallas import tpu_sc as plsc

B, N, K, D_LAT, D_KQ = 2, 10000, 64, 128, 32
P = N * K
NW = 32
CHUNK = 1024
SUB = 128
NSUB = CHUNK // SUB
NCHUNK = 20
PAIRS_PER_W = CHUNK * NCHUNK
P_PAD = NW * PAIRS_PER_W
ROWS_PER_W = PAIRS_PER_W // SUB
SCALE = 1.0 / math.sqrt(D_KQ)


def _proj_body(f_ref, kw_ref, kb_ref, qw_ref, qb_ref, ko_ref, qo_ref):
    f = f_ref[0]
    ko_ref[0] = jnp.dot(f, kw_ref[...], preferred_element_type=jnp.float32) + kb_ref[...]
    qo_ref[0] = jnp.dot(f, qw_ref[...], preferred_element_type=jnp.float32) + qb_ref[...]


def _project(features, ks_w, ks_b, qs_w, qs_b):
    R = 2000
    grid = (B, N // R)
    ks_tab, qs_tab = pl.pallas_call(
        _proj_body,
        grid=grid,
        in_specs=[
            pl.BlockSpec((1, R, D_LAT), lambda b, i: (b, i, 0)),
            pl.BlockSpec((D_LAT, D_KQ), lambda b, i: (0, 0)),
            pl.BlockSpec((1, D_KQ), lambda b, i: (0, 0)),
            pl.BlockSpec((D_LAT, D_KQ), lambda b, i: (0, 0)),
            pl.BlockSpec((1, D_KQ), lambda b, i: (0, 0)),
        ],
        out_specs=[
            pl.BlockSpec((1, R, D_KQ), lambda b, i: (b, i, 0)),
            pl.BlockSpec((1, R, D_KQ), lambda b, i: (b, i, 0)),
        ],
        out_shape=[
            jax.ShapeDtypeStruct((B, N, D_KQ), jnp.float32),
            jax.ShapeDtypeStruct((B, N, D_KQ), jnp.float32),
        ],
    )(features, ks_w.T, ks_b.reshape(1, D_KQ), qs_w.T, qs_b.reshape(1, D_KQ))
    return ks_tab, qs_tab


def _sc_affinity(xidx, yidx, ks0, ks1, qs0, qs1):
    info = plsc.get_sparse_core_info()
    nc = info.num_cores

    mesh = plsc.VectorSubcoreMesh(core_axis_name="c", subcore_axis_name="s")

    @functools.partial(
        pl.kernel,
        mesh=mesh,
        out_type=jax.ShapeDtypeStruct((B, P_PAD), jnp.float32),
        compiler_params=pltpu.CompilerParams(
            needs_layout_passes=False, use_tc_tiling_on_sc=False),
        scratch_types=[
            pltpu.VMEM_SHARED((N, D_KQ), jnp.float32),
            pltpu.VMEM_SHARED((N, D_KQ), jnp.float32),
            pltpu.VMEM((NSUB, SUB), jnp.int32),
            pltpu.VMEM((NSUB, SUB), jnp.int32),
            pltpu.VMEM((CHUNK, D_KQ), jnp.float32),
            pltpu.VMEM((CHUNK, D_KQ), jnp.float32),
            pltpu.VMEM((CHUNK,), jnp.float32),
            pltpu.SemaphoreType.DMA,
            pltpu.SemaphoreType.DMA,
        ],
    )
    def body(xidx_hbm, yidx_hbm, ks0_hbm, ks1_hbm, qs0_hbm, qs1_hbm, out_hbm,
             shk, shq, xidx_v, yidx_v, xrows_v, yrows_v, out_v, sem_x, sem_y):
        wid = lax.axis_index("s") * nc + lax.axis_index("c")
        sid = lax.axis_index("s")
        srow = sid * (N // 16)

        for b, kst, qst in ((0, ks0_hbm, qs0_hbm), (1, ks1_hbm, qs1_hbm)):
            pltpu.sync_copy(kst.at[pl.ds(srow, N // 16)],
                            shk.at[pl.ds(srow, N // 16)])
            pltpu.sync_copy(qst.at[pl.ds(srow, N // 16)],
                            shq.at[pl.ds(srow, N // 16)])
            plsc.subcore_barrier()

            def chunk_body(j, carry):
                rbase = wid * ROWS_PER_W + j * NSUB
                pltpu.sync_copy(xidx_hbm.at[b].at[pl.ds(rbase, NSUB)], xidx_v)
                pltpu.sync_copy(yidx_hbm.at[b].at[pl.ds(rbase, NSUB)], yidx_v)
                copies = []
                for s in range(NSUB):
                    copies.append(pltpu.async_copy(
                        shk.at[xidx_v.at[s]],
                        xrows_v.at[pl.ds(s * SUB, SUB)], sem_x))
                for s in range(NSUB):
                    copies.append(pltpu.async_copy(
                        shq.at[yidx_v.at[s]],
                        yrows_v.at[pl.ds(s * SUB, SUB)], sem_y))
                for c in copies:
                    c.wait()

                def group_body(g, carry2):
                    pids = lax.iota(jnp.int32, 16) + g * 16
                    acc = jnp.zeros((16,), jnp.float32)
                    for d in range(D_KQ):
                        dv = jnp.full((16,), d, jnp.int32)
                        xvv = plsc.load_gather(xrows_v, [pids, dv])
                        yvv = plsc.load_gather(yrows_v, [pids, dv])
                        acc = acc + xvv * yvv
                    off = pl.multiple_of(g * 16, 16)
                    out_v[pl.ds(off, 16)] = acc * SCALE
                    return carry2

                lax.fori_loop(0, CHUNK // 16, group_body, 0)

                pbase = wid * PAIRS_PER_W + j * CHUNK
                pltpu.sync_copy(out_v, out_hbm.at[b].at[pl.ds(pbase, CHUNK)])
                return carry

            lax.fori_loop(0, NCHUNK, chunk_body, 0)
            plsc.subcore_barrier()

    return body(xidx, yidx, ks0, ks1, qs0, qs1)


def kernel(indices, features, ks_w, ks_b, qs_w, qs_b):
    ks_tab, qs_tab = _project(features, ks_w, ks_b, qs_w, qs_b)
    fill = jax.lax.broadcasted_iota(jnp.int32, (B, P_PAD - P), 1) % N
    xidx = jnp.concatenate(
        [indices[1].reshape(B, P).astype(jnp.int32), fill],
        axis=1).reshape(B, P_PAD // SUB, SUB)
    yidx = jnp.concatenate(
        [indices[2].reshape(B, P).astype(jnp.int32), fill],
        axis=1).reshape(B, P_PAD // SUB, SUB)
    out = _sc_affinity(xidx, yidx, ks_tab[0], ks_tab[1], qs_tab[0], qs_tab[1])
    return out[:, :P].reshape(B, N, K)


# speedup vs baseline: 46.3482x; 1.7613x over previous
import functools
import math

import jax
import jax.numpy as jnp
from jax import lax
from jax.experimental import pallas as pl
from jax.experimental.pallas import tpu as pltpu
from jax.experimental.pallas import tpu_sc as plsc

B, N, K, D_LAT, D_KQ = 2, 10000, 64, 128, 32
P = N * K
NW = 32
W = 20480
P_PAD = NW * W
CD = 4
NBLK = D_KQ // CD
CHUNKP = 2048
NCHUNK = W // CHUNKP
TBLK = N * CD
SCALE = 1.0 / math.sqrt(D_KQ)


def _proj_body(f_ref, kw_ref, kb_ref, qw_ref, qb_ref, ko_ref, qo_ref):
    f = f_ref[0]
    ko_ref[0] = jnp.dot(f, kw_ref[...], preferred_element_type=jnp.float32) + kb_ref[...]
    qo_ref[0] = jnp.dot(f, qw_ref[...], preferred_element_type=jnp.float32) + qb_ref[...]


def _project(features, ks_w, ks_b, qs_w, qs_b):
    R = 2000
    grid = (B, N // R)
    ks_tab, qs_tab = pl.pallas_call(
        _proj_body,
        grid=grid,
        in_specs=[
            pl.BlockSpec((1, R, D_LAT), lambda b, i: (b, i, 0)),
            pl.BlockSpec((D_LAT, D_KQ), lambda b, i: (0, 0)),
            pl.BlockSpec((1, D_KQ), lambda b, i: (0, 0)),
            pl.BlockSpec((D_LAT, D_KQ), lambda b, i: (0, 0)),
            pl.BlockSpec((1, D_KQ), lambda b, i: (0, 0)),
        ],
        out_specs=[
            pl.BlockSpec((1, R, D_KQ), lambda b, i: (b, i, 0)),
            pl.BlockSpec((1, R, D_KQ), lambda b, i: (b, i, 0)),
        ],
        out_shape=[
            jax.ShapeDtypeStruct((B, N, D_KQ), jnp.float32),
            jax.ShapeDtypeStruct((B, N, D_KQ), jnp.float32),
        ],
    )(features, ks_w.T, ks_b.reshape(1, D_KQ), qs_w.T, qs_b.reshape(1, D_KQ))
    return ks_tab, qs_tab


def _sc_affinity(xidx, yidx, kb, qb):
    info = plsc.get_sparse_core_info()
    nc = info.num_cores

    mesh = plsc.VectorSubcoreMesh(core_axis_name="c", subcore_axis_name="s")

    @functools.partial(
        pl.kernel,
        mesh=mesh,
        out_type=jax.ShapeDtypeStruct((B, P_PAD), jnp.float32),
        compiler_params=pltpu.CompilerParams(
            needs_layout_passes=False, use_tc_tiling_on_sc=False),
        scratch_types=[
            pltpu.VMEM((TBLK,), jnp.float32),
            pltpu.VMEM((TBLK,), jnp.float32),
            pltpu.VMEM((2, CHUNKP), jnp.int32),
            pltpu.VMEM((2, CHUNKP), jnp.int32),
            pltpu.VMEM((W,), jnp.float32),
            pltpu.SemaphoreType.DMA,
            pltpu.SemaphoreType.DMA,
        ],
    )
    def body(xi_hbm, yi_hbm, kb_hbm, qb_hbm, out_hbm,
             kv, qv, xiv, yiv, acc, sem_t, sem_i):
        wid = lax.axis_index("s") * nc + lax.axis_index("c")
        pbase0 = wid * W

        for b in (0, 1):
            for blk in range(NBLK):
                tbase = (b * NBLK + blk) * TBLK
                ck = pltpu.async_copy(kb_hbm.at[pl.ds(tbase, TBLK)], kv, sem_t)
                cq = pltpu.async_copy(qb_hbm.at[pl.ds(tbase, TBLK)], qv, sem_t)
                c0 = pltpu.async_copy(
                    xi_hbm.at[b].at[pl.ds(pbase0, CHUNKP)], xiv.at[0], sem_i)
                c1 = pltpu.async_copy(
                    yi_hbm.at[b].at[pl.ds(pbase0, CHUNKP)], yiv.at[0], sem_i)
                ck.wait(); cq.wait(); c0.wait(); c1.wait()

                def chunk_body(j, carry):
                    nxt = (j + 1) % 2
                    cur = j % 2

                    @pl.when(j + 1 < NCHUNK)
                    def _():
                        pbn = pbase0 + (j + 1) * CHUNKP
                        pltpu.async_copy(
                            xi_hbm.at[b].at[pl.ds(pbn, CHUNKP)],
                            xiv.at[nxt], sem_i)
                        pltpu.async_copy(
                            yi_hbm.at[b].at[pl.ds(pbn, CHUNKP)],
                            yiv.at[nxt], sem_i)

                    def group(g, carry2):
                        off = pl.multiple_of(g * 16, 16)
                        xv16 = xiv[cur, pl.ds(off, 16)]
                        yv16 = yiv[cur, pl.ds(off, 16)]
                        aoff = j * CHUNKP + off
                        a = acc[pl.ds(aoff, 16)] if blk > 0 \
                            else jnp.zeros((16,), jnp.float32)
                        xbase = xv16 * CD
                        ybase = yv16 * CD
                        for d in range(CD):
                            kvv = plsc.load_gather(kv, [xbase + d])
                            qvv = plsc.load_gather(qv, [ybase + d])
                            a = a + kvv * qvv
                        if blk == NBLK - 1:
                            a = a * SCALE
                        acc[pl.ds(aoff, 16)] = a
                        return carry2

                    lax.fori_loop(0, CHUNKP // 16, group, 0)

                    @pl.when(j + 1 < NCHUNK)
                    def _():
                        pltpu.make_async_copy(
                            xi_hbm.at[b].at[pl.ds(pbase0, CHUNKP)],
                            xiv.at[nxt], sem_i).wait()
                        pltpu.make_async_copy(
                            yi_hbm.at[b].at[pl.ds(pbase0, CHUNKP)],
                            yiv.at[nxt], sem_i).wait()
                    return carry

                lax.fori_loop(0, NCHUNK, chunk_body, 0)
            pltpu.sync_copy(acc, out_hbm.at[b].at[pl.ds(pbase0, W)])

    return body(xidx, yidx, kb, qb)


def kernel(indices, features, ks_w, ks_b, qs_w, qs_b):
    ks_tab, qs_tab = _project(features, ks_w, ks_b, qs_w, qs_b)
    kb = ks_tab.reshape(B, N, NBLK, CD).transpose(0, 2, 1, 3).reshape(-1)
    qb = qs_tab.reshape(B, N, NBLK, CD).transpose(0, 2, 1, 3).reshape(-1)
    fill = jax.lax.broadcasted_iota(jnp.int32, (B, P_PAD - P), 1) % N
    xidx = jnp.concatenate(
        [indices[1].reshape(B, P).astype(jnp.int32), fill], axis=1)
    yidx = jnp.concatenate(
        [indices[2].reshape(B, P).astype(jnp.int32), fill], axis=1)
    out = _sc_affinity(xidx, yidx, kb, qb)
    return out[:, :P].reshape(B, N, K)


# speedup vs baseline: 46.8501x; 1.0108x over previous
import functools
import math

import jax
import jax.numpy as jnp
from jax import lax
from jax.experimental import pallas as pl
from jax.experimental.pallas import tpu as pltpu
from jax.experimental.pallas import tpu_sc as plsc

B, N, K, D_LAT, D_KQ = 2, 10000, 64, 128, 32
P = N * K
NW = 32
W = 20480
P_PAD = NW * W
CD = 4
NBLK = D_KQ // CD
CHUNKP = 2048
NCHUNK = W // CHUNKP
TBLK = N * CD
SCALE = 1.0 / math.sqrt(D_KQ)


def _proj_body(f_ref, kw_ref, kb_ref, qw_ref, qb_ref, ko_ref, qo_ref):
    f = f_ref[0]
    ko_ref[0] = jnp.dot(f, kw_ref[...], preferred_element_type=jnp.float32) + kb_ref[...]
    qo_ref[0] = jnp.dot(f, qw_ref[...], preferred_element_type=jnp.float32) + qb_ref[...]


def _project(features, ks_w, ks_b, qs_w, qs_b):
    R = 2000
    grid = (B, N // R)
    ks_tab, qs_tab = pl.pallas_call(
        _proj_body,
        grid=grid,
        in_specs=[
            pl.BlockSpec((1, R, D_LAT), lambda b, i: (b, i, 0)),
            pl.BlockSpec((D_LAT, D_KQ), lambda b, i: (0, 0)),
            pl.BlockSpec((1, D_KQ), lambda b, i: (0, 0)),
            pl.BlockSpec((D_LAT, D_KQ), lambda b, i: (0, 0)),
            pl.BlockSpec((1, D_KQ), lambda b, i: (0, 0)),
        ],
        out_specs=[
            pl.BlockSpec((1, R, D_KQ), lambda b, i: (b, i, 0)),
            pl.BlockSpec((1, R, D_KQ), lambda b, i: (b, i, 0)),
        ],
        out_shape=[
            jax.ShapeDtypeStruct((B, N, D_KQ), jnp.float32),
            jax.ShapeDtypeStruct((B, N, D_KQ), jnp.float32),
        ],
    )(features, ks_w.T, ks_b.reshape(1, D_KQ), qs_w.T, qs_b.reshape(1, D_KQ))
    return ks_tab, qs_tab


def _sc_affinity(xidx, yidx, kb, qb):
    info = plsc.get_sparse_core_info()
    nc = info.num_cores

    mesh = plsc.VectorSubcoreMesh(core_axis_name="c", subcore_axis_name="s")

    @functools.partial(
        pl.kernel,
        mesh=mesh,
        out_type=jax.ShapeDtypeStruct((B, P_PAD), jnp.float32),
        compiler_params=pltpu.CompilerParams(
            needs_layout_passes=False, use_tc_tiling_on_sc=False),
        scratch_types=[
            pltpu.VMEM((TBLK,), jnp.float32),
            pltpu.VMEM((TBLK,), jnp.float32),
            pltpu.VMEM((2, CHUNKP), jnp.int32),
            pltpu.VMEM((2, CHUNKP), jnp.int32),
            pltpu.VMEM((W,), jnp.float32),
            pltpu.SemaphoreType.DMA,
            pltpu.SemaphoreType.DMA,
        ],
    )
    def body(xi_hbm, yi_hbm, kb_hbm, qb_hbm, out_hbm,
             kv, qv, xiv, yiv, acc, sem_t, sem_i):
        wid = lax.axis_index("s") * nc + lax.axis_index("c")
        pbase0 = wid * W

        for b in (0, 1):
            for blk in range(NBLK):
                tbase = (b * NBLK + blk) * TBLK
                ck = pltpu.async_copy(kb_hbm.at[pl.ds(tbase, TBLK)], kv, sem_t)
                cq = pltpu.async_copy(qb_hbm.at[pl.ds(tbase, TBLK)], qv, sem_t)
                c0 = pltpu.async_copy(
                    xi_hbm.at[b].at[pl.ds(pbase0, CHUNKP)], xiv.at[0], sem_i)
                c1 = pltpu.async_copy(
                    yi_hbm.at[b].at[pl.ds(pbase0, CHUNKP)], yiv.at[0], sem_i)
                ck.wait(); cq.wait(); c0.wait(); c1.wait()

                def chunk_body(j, carry):
                    nxt = (j + 1) % 2
                    cur = j % 2

                    @pl.when(j + 1 < NCHUNK)
                    def _():
                        pbn = pbase0 + (j + 1) * CHUNKP
                        pltpu.async_copy(
                            xi_hbm.at[b].at[pl.ds(pbn, CHUNKP)],
                            xiv.at[nxt], sem_i)
                        pltpu.async_copy(
                            yi_hbm.at[b].at[pl.ds(pbn, CHUNKP)],
                            yiv.at[nxt], sem_i)

                    def group(g, carry2):
                        gbase = pl.multiple_of(g * 64, 64)
                        for u in range(4):
                            off = gbase + u * 16
                            xv16 = xiv[cur, pl.ds(off, 16)]
                            yv16 = yiv[cur, pl.ds(off, 16)]
                            aoff = j * CHUNKP + off
                            a = acc[pl.ds(aoff, 16)] if blk > 0 \
                                else jnp.zeros((16,), jnp.float32)
                            xbase = xv16 * CD
                            ybase = yv16 * CD
                            for d in range(CD):
                                kvv = plsc.load_gather(kv, [xbase + d])
                                qvv = plsc.load_gather(qv, [ybase + d])
                                a = a + kvv * qvv
                            if blk == NBLK - 1:
                                a = a * SCALE
                            acc[pl.ds(aoff, 16)] = a
                        return carry2

                    lax.fori_loop(0, CHUNKP // 64, group, 0)

                    @pl.when(j + 1 < NCHUNK)
                    def _():
                        pltpu.make_async_copy(
                            xi_hbm.at[b].at[pl.ds(pbase0, CHUNKP)],
                            xiv.at[nxt], sem_i).wait()
                        pltpu.make_async_copy(
                            yi_hbm.at[b].at[pl.ds(pbase0, CHUNKP)],
                            yiv.at[nxt], sem_i).wait()
                    return carry

                lax.fori_loop(0, NCHUNK, chunk_body, 0)
            pltpu.sync_copy(acc, out_hbm.at[b].at[pl.ds(pbase0, W)])

    return body(xidx, yidx, kb, qb)


def kernel(indices, features, ks_w, ks_b, qs_w, qs_b):
    ks_tab, qs_tab = _project(features, ks_w, ks_b, qs_w, qs_b)
    kb = ks_tab.reshape(B, N, NBLK, CD).transpose(0, 2, 1, 3).reshape(-1)
    qb = qs_tab.reshape(B, N, NBLK, CD).transpose(0, 2, 1, 3).reshape(-1)
    fill = jax.lax.broadcasted_iota(jnp.int32, (B, P_PAD - P), 1) % N
    xidx = jnp.concatenate(
        [indices[1].reshape(B, P).astype(jnp.int32), fill], axis=1)
    yidx = jnp.concatenate(
        [indices[2].reshape(B, P).astype(jnp.int32), fill], axis=1)
    out = _sc_affinity(xidx, yidx, kb, qb)
    return out[:, :P].reshape(B, N, K)


# speedup vs baseline: 70.0988x; 1.4962x over previous
import functools
import math

import jax
import jax.numpy as jnp
from jax import lax
from jax.experimental import pallas as pl
from jax.experimental.pallas import tpu as pltpu
from jax.experimental.pallas import tpu_sc as plsc

B, N, K, D_LAT, D_KQ = 2, 10000, 64, 128, 32
P = N * K
NW = 32
W = 20480
P_PAD = NW * W
CD = 4
NBLK = D_KQ // CD
CHUNKP = 2048
NCHUNK = W // CHUNKP
TBLK = N * CD
SCALE = 1.0 / math.sqrt(D_KQ)


def _proj_body(f_ref, kw_ref, kb_ref, qw_ref, qb_ref, ko_ref, qo_ref):
    f = f_ref[0]
    ko_ref[0] = jnp.dot(f, kw_ref[...], preferred_element_type=jnp.float32) + kb_ref[...]
    qo_ref[0] = jnp.dot(f, qw_ref[...], preferred_element_type=jnp.float32) + qb_ref[...]


def _project(features, ks_w, ks_b, qs_w, qs_b):
    R = 2000
    grid = (B, N // R)
    ks_tab, qs_tab = pl.pallas_call(
        _proj_body,
        grid=grid,
        in_specs=[
            pl.BlockSpec((1, R, D_LAT), lambda b, i: (b, i, 0)),
            pl.BlockSpec((D_LAT, D_KQ), lambda b, i: (0, 0)),
            pl.BlockSpec((1, D_KQ), lambda b, i: (0, 0)),
            pl.BlockSpec((D_LAT, D_KQ), lambda b, i: (0, 0)),
            pl.BlockSpec((1, D_KQ), lambda b, i: (0, 0)),
        ],
        out_specs=[
            pl.BlockSpec((1, R, D_KQ), lambda b, i: (b, i, 0)),
            pl.BlockSpec((1, R, D_KQ), lambda b, i: (b, i, 0)),
        ],
        out_shape=[
            jax.ShapeDtypeStruct((B, N, D_KQ), jnp.float32),
            jax.ShapeDtypeStruct((B, N, D_KQ), jnp.float32),
        ],
    )(features, ks_w.T, ks_b.reshape(1, D_KQ), qs_w.T, qs_b.reshape(1, D_KQ))
    return ks_tab, qs_tab


def _sc_affinity(xidx, yidx, kb, qb):
    info = plsc.get_sparse_core_info()
    nc = info.num_cores

    mesh = plsc.VectorSubcoreMesh(core_axis_name="c", subcore_axis_name="s")

    @functools.partial(
        pl.kernel,
        mesh=mesh,
        out_type=jax.ShapeDtypeStruct((B, P_PAD), jnp.float32),
        compiler_params=pltpu.CompilerParams(
            needs_layout_passes=False, use_tc_tiling_on_sc=False),
        scratch_types=[
            pltpu.VMEM((TBLK,), jnp.float32),
            pltpu.VMEM((TBLK,), jnp.float32),
            pltpu.VMEM((2, CHUNKP), jnp.int32),
            pltpu.VMEM((2, CHUNKP), jnp.int32),
            pltpu.VMEM((W,), jnp.float32),
            pltpu.SemaphoreType.DMA,
            pltpu.SemaphoreType.DMA,
        ],
    )
    def body(xi_hbm, yi_hbm, kb_hbm, qb_hbm, out_hbm,
             kv, qv, xiv, yiv, acc, sem_t, sem_i):
        wid = lax.axis_index("s") * nc + lax.axis_index("c")
        pbase0 = wid * W

        for b in (0, 1):
            for blk in range(NBLK):
                tbase = (b * NBLK + blk) * TBLK
                ck = pltpu.async_copy(kb_hbm.at[pl.ds(tbase, TBLK)], kv, sem_t)
                cq = pltpu.async_copy(qb_hbm.at[pl.ds(tbase, TBLK)], qv, sem_t)
                c0 = pltpu.async_copy(
                    xi_hbm.at[b].at[pl.ds(pbase0, CHUNKP)], xiv.at[0], sem_i)
                c1 = pltpu.async_copy(
                    yi_hbm.at[b].at[pl.ds(pbase0, CHUNKP)], yiv.at[0], sem_i)
                ck.wait(); cq.wait(); c0.wait(); c1.wait()

                def chunk_body(j, carry):
                    nxt = (j + 1) % 2
                    cur = j % 2

                    @pl.when(j + 1 < NCHUNK)
                    def _():
                        pbn = pbase0 + (j + 1) * CHUNKP
                        pltpu.async_copy(
                            xi_hbm.at[b].at[pl.ds(pbn, CHUNKP)],
                            xiv.at[nxt], sem_i)
                        pltpu.async_copy(
                            yi_hbm.at[b].at[pl.ds(pbn, CHUNKP)],
                            yiv.at[nxt], sem_i)

                    def group(g, carry2):
                        gbase = pl.multiple_of(g * 64, 64)
                        for u in range(4):
                            off = gbase + u * 16
                            xv16 = xiv[cur, pl.ds(off, 16)]
                            yv16 = yiv[cur, pl.ds(off, 16)]
                            aoff = j * CHUNKP + off
                            a = acc[pl.ds(aoff, 16)] if blk > 0 \
                                else jnp.zeros((16,), jnp.float32)
                            for d in range(CD):
                                kvv = plsc.load_gather(kv, [xv16 + d * N])
                                qvv = plsc.load_gather(qv, [yv16 + d * N])
                                a = a + kvv * qvv
                            if blk == NBLK - 1:
                                a = a * SCALE
                            acc[pl.ds(aoff, 16)] = a
                        return carry2

                    lax.fori_loop(0, CHUNKP // 64, group, 0)

                    @pl.when(j + 1 < NCHUNK)
                    def _():
                        pltpu.make_async_copy(
                            xi_hbm.at[b].at[pl.ds(pbase0, CHUNKP)],
                            xiv.at[nxt], sem_i).wait()
                        pltpu.make_async_copy(
                            yi_hbm.at[b].at[pl.ds(pbase0, CHUNKP)],
                            yiv.at[nxt], sem_i).wait()
                    return carry

                lax.fori_loop(0, NCHUNK, chunk_body, 0)
            pltpu.sync_copy(acc, out_hbm.at[b].at[pl.ds(pbase0, W)])

    return body(xidx, yidx, kb, qb)


def kernel(indices, features, ks_w, ks_b, qs_w, qs_b):
    ks_tab, qs_tab = _project(features, ks_w, ks_b, qs_w, qs_b)
    kb = ks_tab.reshape(B, N, NBLK, CD).transpose(0, 2, 3, 1).reshape(-1)
    qb = qs_tab.reshape(B, N, NBLK, CD).transpose(0, 2, 3, 1).reshape(-1)
    fill = jax.lax.broadcasted_iota(jnp.int32, (B, P_PAD - P), 1) % N
    xidx = jnp.concatenate(
        [indices[1].reshape(B, P).astype(jnp.int32), fill], axis=1)
    yidx = jnp.concatenate(
        [indices[2].reshape(B, P).astype(jnp.int32), fill], axis=1)
    out = _sc_affinity(xidx, yidx, kb, qb)
    return out[:, :P].reshape(B, N, K)
